# baseline (device time: 180448 ns/iter reference)
import jax
import jax.numpy as jnp
from jax import lax
from jax.experimental import pallas as pl
from jax.experimental.pallas import tpu as pltpu

N_DEV = 16
SQ = 1024
SKV = 1024
DH = 128
H_LOC = 8
BLK = 64
CHUNK = SQ // N_DEV
SCALE = 0.08838834764831843


def _body(x_ref, wq_ref, k_ref, v_ref, wo_ref, out_ref,
          q_ref, ctx_ref, bias_ref, comm_ref,
          rs_send, rs_recv, ag_send, ag_recv):
    my = lax.axis_index("i")
    left = lax.rem(my - 1 + N_DEV, N_DEV)
    right = lax.rem(my + 1, N_DEV)

    barrier = pltpu.get_barrier_semaphore()
    for nbr in (left, right):
        pl.semaphore_signal(barrier, inc=1, device_id=(nbr,),
                            device_id_type=pl.DeviceIdType.MESH)
    pl.semaphore_wait(barrier, 2)

    q_ref[:, :] = lax.dot_general(
        x_ref[:, :], wq_ref[:, :], (((1,), (0,)), ((), ())),
        preferred_element_type=jnp.float32).astype(jnp.bfloat16)

    r = lax.broadcasted_iota(jnp.int32, (SQ, SKV), 0) // BLK
    c = lax.broadcasted_iota(jnp.int32, (SQ, SKV), 1) // BLK
    mask = (r == c) | (c == 0) | (lax.rem(r + c, 3) == 0)
    bias_ref[:, :] = jnp.where(mask, 0.0, -1e9)

    for h in range(H_LOC):
        sl = slice(h * DH, (h + 1) * DH)
        qh = q_ref[:, sl]
        kh = k_ref[:, sl]
        vh = v_ref[:, sl]
        scores = lax.dot_general(qh, kh, (((1,), (1,)), ((), ())),
                                 preferred_element_type=jnp.float32)
        scores = scores * SCALE + bias_ref[:, :]
        mx = jnp.max(scores, axis=1, keepdims=True)
        e = jnp.exp(scores - mx)
        s = jnp.sum(e, axis=1, keepdims=True)
        w = (e / s).astype(jnp.bfloat16)
        ctx_ref[:, sl] = lax.dot_general(
            w, vh, (((1,), (0,)), ((), ())),
            preferred_element_type=jnp.float32).astype(jnp.bfloat16)

    out_ref[:, :] = lax.dot_general(
        ctx_ref[:, :], wo_ref[:, :], (((1,), (0,)), ((), ())),
        preferred_element_type=jnp.float32)

    for s in range(N_DEV - 1):
        send_chunk = lax.rem(my - s + N_DEV, N_DEV)
        if s == 0:
            src = out_ref.at[pl.ds(send_chunk * CHUNK, CHUNK), :]
        else:
            src = comm_ref.at[s - 1]
        rdma = pltpu.make_async_remote_copy(
            src_ref=src,
            dst_ref=comm_ref.at[s],
            send_sem=rs_send.at[s],
            recv_sem=rs_recv.at[s],
            device_id=(right,),
            device_id_type=pl.DeviceIdType.MESH,
        )
        rdma.start()
        rdma.wait()
        recv_chunk = lax.rem(my - s - 1 + N_DEV, N_DEV)
        comm_ref[s, :, :] = (comm_ref[s, :, :]
                             + out_ref[pl.ds(recv_chunk * CHUNK, CHUNK), :])

    red_chunk = lax.rem(my + 1, N_DEV)
    out_ref[pl.ds(red_chunk * CHUNK, CHUNK), :] = comm_ref[N_DEV - 2, :, :]

    for t in range(N_DEV - 1):
        send_chunk = lax.rem(my + 1 - t + N_DEV, N_DEV)
        rdma = pltpu.make_async_remote_copy(
            src_ref=out_ref.at[pl.ds(send_chunk * CHUNK, CHUNK), :],
            dst_ref=out_ref.at[pl.ds(send_chunk * CHUNK, CHUNK), :],
            send_sem=ag_send.at[t],
            recv_sem=ag_recv.at[t],
            device_id=(right,),
            device_id_type=pl.DeviceIdType.MESH,
        )
        rdma.start()
        rdma.wait()


def kernel(x, Wq, K_ext, V_ext, Wo):
    my = lax.axis_index("i")
    x2 = x.reshape(SQ, 1024).astype(jnp.bfloat16)
    wq = Wq.astype(jnp.bfloat16)
    k = lax.dynamic_slice(
        K_ext, (0, 0, my * H_LOC, 0), (1, SKV, H_LOC, DH)
    ).reshape(SKV, H_LOC * DH).astype(jnp.bfloat16)
    v = lax.dynamic_slice(
        V_ext, (0, 0, my * H_LOC, 0), (1, SKV, H_LOC, DH)
    ).reshape(SKV, H_LOC * DH).astype(jnp.bfloat16)
    wo = Wo.astype(jnp.bfloat16)

    out = pl.pallas_call(
        _body,
        out_shape=jax.ShapeDtypeStruct((SQ, 1024), jnp.float32),
        in_specs=[pl.BlockSpec(memory_space=pltpu.VMEM)] * 5,
        out_specs=pl.BlockSpec(memory_space=pltpu.VMEM),
        scratch_shapes=[
            pltpu.VMEM((SQ, H_LOC * DH), jnp.bfloat16),
            pltpu.VMEM((SQ, H_LOC * DH), jnp.bfloat16),
            pltpu.VMEM((SQ, SKV), jnp.float32),
            pltpu.VMEM((N_DEV - 1, CHUNK, 1024), jnp.float32),
            pltpu.SemaphoreType.DMA((N_DEV - 1,)),
            pltpu.SemaphoreType.DMA((N_DEV - 1,)),
            pltpu.SemaphoreType.DMA((N_DEV - 1,)),
            pltpu.SemaphoreType.DMA((N_DEV - 1,)),
        ],
        compiler_params=pltpu.CompilerParams(collective_id=0),
    )(x2, wq, k, v, wo)
    return out.reshape(1, SQ, 1024)


# device time: 85728 ns/iter; 2.1049x vs baseline; 2.1049x over previous
import jax
import jax.numpy as jnp
from jax import lax
from jax.experimental import pallas as pl
from jax.experimental.pallas import tpu as pltpu

N_DEV = 16
SQ = 1024
SKV = 1024
DH = 128
H_LOC = 8
BLK = 64
CHUNK = SQ // N_DEV
SCALE = 0.08838834764831843


def _body(x_ref, wq_ref, k_ref, v_ref, wo_ref, out_ref,
          q_ref, ctx_ref, bias_ref, part_ref, rs_buf,
          rs_send, rs_recv, ag_send, ag_recv):
    my = lax.axis_index("i")

    barrier = pltpu.get_barrier_semaphore()
    for j in range(1, N_DEV):
        tgt = lax.rem(my + j, N_DEV)
        pl.semaphore_signal(barrier, inc=1, device_id=(tgt,),
                            device_id_type=pl.DeviceIdType.MESH)
    pl.semaphore_wait(barrier, N_DEV - 1)

    q_ref[:, :] = lax.dot_general(
        x_ref[:, :], wq_ref[:, :], (((1,), (0,)), ((), ())),
        preferred_element_type=jnp.float32).astype(jnp.bfloat16)

    r = lax.broadcasted_iota(jnp.int32, (SQ, SKV), 0) // BLK
    c = lax.broadcasted_iota(jnp.int32, (SQ, SKV), 1) // BLK
    mask = (r == c) | (c == 0) | (lax.rem(r + c, 3) == 0)
    bias_ref[:, :] = jnp.where(mask, 0.0, -1e9)

    for h in range(H_LOC):
        sl = slice(h * DH, (h + 1) * DH)
        qh = q_ref[:, sl]
        kh = k_ref[:, sl]
        vh = v_ref[:, sl]
        scores = lax.dot_general(qh, kh, (((1,), (1,)), ((), ())),
                                 preferred_element_type=jnp.float32)
        scores = scores * SCALE + bias_ref[:, :]
        mx = jnp.max(scores, axis=1, keepdims=True)
        e = jnp.exp(scores - mx)
        s = jnp.sum(e, axis=1, keepdims=True)
        w = (e / s).astype(jnp.bfloat16)
        ctx_ref[:, sl] = lax.dot_general(
            w, vh, (((1,), (0,)), ((), ())),
            preferred_element_type=jnp.float32).astype(jnp.bfloat16)

    part_ref[:, :] = lax.dot_general(
        ctx_ref[:, :], wo_ref[:, :], (((1,), (0,)), ((), ())),
        preferred_element_type=jnp.float32).astype(jnp.bfloat16)

    rs_rdmas = []
    for j in range(1, N_DEV):
        tgt = lax.rem(my + j, N_DEV)
        rdma = pltpu.make_async_remote_copy(
            src_ref=part_ref.at[pl.ds(tgt * CHUNK, CHUNK), :],
            dst_ref=rs_buf.at[j - 1],
            send_sem=rs_send.at[j - 1],
            recv_sem=rs_recv.at[j - 1],
            device_id=(tgt,),
            device_id_type=pl.DeviceIdType.MESH,
        )
        rdma.start()
        rs_rdmas.append(rdma)

    for rdma in rs_rdmas:
        rdma.wait_recv()

    acc = part_ref[pl.ds(my * CHUNK, CHUNK), :].astype(jnp.float32)
    for k in range(N_DEV - 1):
        acc = acc + rs_buf[k, :, :].astype(jnp.float32)
    out_ref[pl.ds(my * CHUNK, CHUNK), :] = acc.astype(jnp.bfloat16)

    ag_rdmas = []
    for j in range(1, N_DEV):
        tgt = lax.rem(my + j, N_DEV)
        rdma = pltpu.make_async_remote_copy(
            src_ref=out_ref.at[pl.ds(my * CHUNK, CHUNK), :],
            dst_ref=out_ref.at[pl.ds(my * CHUNK, CHUNK), :],
            send_sem=ag_send.at[j - 1],
            recv_sem=ag_recv.at[j - 1],
            device_id=(tgt,),
            device_id_type=pl.DeviceIdType.MESH,
        )
        rdma.start()
        ag_rdmas.append(rdma)

    for k in range(N_DEV - 1):
        src_dev = lax.rem(my - k - 1 + N_DEV, N_DEV)
        recv = pltpu.make_async_remote_copy(
            src_ref=out_ref.at[pl.ds(my * CHUNK, CHUNK), :],
            dst_ref=out_ref.at[pl.ds(src_dev * CHUNK, CHUNK), :],
            send_sem=ag_send.at[k],
            recv_sem=ag_recv.at[k],
            device_id=(my,),
            device_id_type=pl.DeviceIdType.MESH,
        )
        recv.wait_recv()

    for rdma in rs_rdmas:
        rdma.wait_send()
    for rdma in ag_rdmas:
        rdma.wait_send()


def kernel(x, Wq, K_ext, V_ext, Wo):
    my = lax.axis_index("i")
    x2 = x.reshape(SQ, 1024).astype(jnp.bfloat16)
    wq = Wq.astype(jnp.bfloat16)
    k = lax.dynamic_slice(
        K_ext, (0, 0, my * H_LOC, 0), (1, SKV, H_LOC, DH)
    ).reshape(SKV, H_LOC * DH).astype(jnp.bfloat16)
    v = lax.dynamic_slice(
        V_ext, (0, 0, my * H_LOC, 0), (1, SKV, H_LOC, DH)
    ).reshape(SKV, H_LOC * DH).astype(jnp.bfloat16)
    wo = Wo.astype(jnp.bfloat16)

    out = pl.pallas_call(
        _body,
        out_shape=jax.ShapeDtypeStruct((SQ, 1024), jnp.bfloat16),
        in_specs=[pl.BlockSpec(memory_space=pltpu.VMEM)] * 5,
        out_specs=pl.BlockSpec(memory_space=pltpu.VMEM),
        scratch_shapes=[
            pltpu.VMEM((SQ, H_LOC * DH), jnp.bfloat16),
            pltpu.VMEM((SQ, H_LOC * DH), jnp.bfloat16),
            pltpu.VMEM((SQ, SKV), jnp.float32),
            pltpu.VMEM((SQ, 1024), jnp.bfloat16),
            pltpu.VMEM((N_DEV - 1, CHUNK, 1024), jnp.bfloat16),
            pltpu.SemaphoreType.DMA((N_DEV - 1,)),
            pltpu.SemaphoreType.DMA((N_DEV - 1,)),
            pltpu.SemaphoreType.DMA((N_DEV - 1,)),
            pltpu.SemaphoreType.DMA((N_DEV - 1,)),
        ],
        compiler_params=pltpu.CompilerParams(collective_id=0),
    )(x2, wq, k, v, wo)
    return out.reshape(1, SQ, 1024)
